# Initial kernel scaffold; baseline (speedup 1.0000x reference)
import jax
import jax.numpy as jnp
from jax.experimental import pallas as pl


def kernel(t0, t1, t2, t3, t4, t5, t6):
    raise NotImplementedError("write your pallas kernel here")


# speedup vs baseline: 4.0241x; 4.0241x over previous
import functools

import jax
import jax.numpy as jnp
from jax import lax
from jax.experimental import pallas as pl
from jax.experimental.pallas import tpu as pltpu
from jax.experimental.pallas import tpu_sc as plsc

N = 100000
D = 128
S = 512
NC = 2
NS = 16
NW = NC * NS
CHUNK = 128
TOT_CHUNKS = 800
CPW = TOT_CHUNKS // NW
FULL_CHUNKS = N // CHUNK
REM = N - FULL_CHUNKS * CHUNK


def _seg_sum_body(x_hbm, ids_hbm, sums_hbm, cnts_hbm,
                  idx_v, buf_v, ones_v, acc_sh, cnt_sh):
    cid = lax.axis_index("c")
    sid = lax.axis_index("s")
    w = sid * NC + cid

    zvec = jnp.zeros((16,), jnp.float32)
    onevec = jnp.ones((16,), jnp.float32)

    @pl.loop(0, 33)
    def _(r):
        for k in range(D // 16):
            buf_v[r, pl.ds(16 * k, 16)] = zvec

    @pl.loop(0, CHUNK)
    def _(r):
        for k in range(D // 16):
            ones_v[r, pl.ds(16 * k, 16)] = onevec

    pltpu.sync_copy(buf_v.at[pl.ds(0, 32)], acc_sh.at[pl.ds(32 * sid, 32)])
    pltpu.sync_copy(buf_v.at[pl.ds(0, 32)], cnt_sh.at[pl.ds(32 * sid, 32)])

    @pl.when(sid == 0)
    def _():
        pltpu.sync_copy(buf_v.at[pl.ds(0, 1)], acc_sh.at[pl.ds(S, 1)])
        pltpu.sync_copy(buf_v.at[pl.ds(0, 1)], cnt_sh.at[pl.ds(S, 1)])

    pltpu.sync_copy(ids_hbm.at[w], idx_v)

    plsc.subcore_barrier()

    @pl.loop(0, CPW)
    def _(j):
        c = w * CPW + j
        row0 = c * CHUNK

        @pl.when(c < FULL_CHUNKS)
        def _():
            pltpu.sync_copy(x_hbm.at[pl.ds(row0, CHUNK)], buf_v)

        @pl.when(c == FULL_CHUNKS)
        def _():
            pltpu.sync_copy(x_hbm.at[pl.ds(row0, REM)], buf_v.at[pl.ds(0, REM)])

        @pl.when(c <= FULL_CHUNKS)
        def _():
            pltpu.sync_copy(buf_v, acc_sh.at[idx_v.at[j]], add=True)
            pltpu.sync_copy(ones_v, cnt_sh.at[idx_v.at[j]], add=True)

    plsc.subcore_barrier()

    pltpu.sync_copy(acc_sh.at[pl.ds(32 * sid, 32)],
                    sums_hbm.at[cid, pl.ds(32 * sid, 32)])
    pltpu.sync_copy(cnt_sh.at[pl.ds(32 * sid, 32)],
                    cnts_hbm.at[cid, pl.ds(32 * sid, 32)])


@jax.jit
def _seg_sum(x, ids2d):
    return pl.kernel(
        _seg_sum_body,
        out_type=[
            jax.ShapeDtypeStruct((NC, S, D), jnp.float32),
            jax.ShapeDtypeStruct((NC, S, D), jnp.float32),
        ],
        mesh=plsc.VectorSubcoreMesh(
            core_axis_name="c", subcore_axis_name="s",
            num_cores=NC, num_subcores=NS),
        scratch_types=[
            pltpu.VMEM((CPW, CHUNK), jnp.int32),
            pltpu.VMEM((CHUNK, D), jnp.float32),
            pltpu.VMEM((CHUNK, D), jnp.float32),
            pltpu.VMEM_SHARED((S + 1, D), jnp.float32),
            pltpu.VMEM_SHARED((S + 1, D), jnp.float32),
        ],
    )(x, ids2d)


def _combine_body(sums_ref, cnts_ref, out_ref):
    s = sums_ref[0] + sums_ref[1]
    c = cnts_ref[0] + cnts_ref[1]
    out_ref[...] = s / jnp.maximum(c, 1.0)


@jax.jit
def _combine(sums, cnts):
    return pl.pallas_call(
        _combine_body,
        out_shape=jax.ShapeDtypeStruct((S, D), jnp.float32),
    )(sums, cnts)


def kernel(t0, t1, t2, t3, t4, t5, t6):
    ids = t4.astype(jnp.int32)
    pad = jnp.full((TOT_CHUNKS * CHUNK - N,), S, dtype=jnp.int32)
    ids3d = jnp.concatenate([ids, pad]).reshape(NW, CPW, CHUNK)
    sums, cnts = _seg_sum(t0, ids3d)
    x_graph = _combine(sums, cnts)
    return (t0, t1, t2, t3, t4, x_graph, t6)


# speedup vs baseline: 4.7281x; 1.1749x over previous
import functools

import jax
import jax.numpy as jnp
from jax import lax
from jax.experimental import pallas as pl
from jax.experimental.pallas import tpu as pltpu
from jax.experimental.pallas import tpu_sc as plsc

N = 100000
D = 128
S = 512
NC = 2
NS = 16
NW = NC * NS
CHUNK = 128
TOT_CHUNKS = 800
CPW = TOT_CHUNKS // NW
FULL_CHUNKS = N // CHUNK
REM = N - FULL_CHUNKS * CHUNK


def _seg_sum_body(x_hbm, ids_hbm, sums_hbm, cnts_hbm,
                  idx_v, buf_a, buf_b, ones_v, acc_sh, cnt_sh, sem_a, sem_b):
    cid = lax.axis_index("c")
    sid = lax.axis_index("s")
    w = sid * NC + cid

    zvec = jnp.zeros((16,), jnp.float32)
    onevec = jnp.ones((16,), jnp.float32)

    @pl.loop(0, 33)
    def _(r):
        for k in range(D // 16):
            buf_a[r, pl.ds(16 * k, 16)] = zvec

    @pl.loop(0, CHUNK)
    def _(r):
        for k in range(D // 16):
            ones_v[r, pl.ds(16 * k, 16)] = onevec

    pltpu.sync_copy(buf_a.at[pl.ds(0, 32)], acc_sh.at[pl.ds(32 * sid, 32)])
    pltpu.sync_copy(buf_a.at[pl.ds(0, 32)], cnt_sh.at[pl.ds(32 * sid, 32)])

    @pl.when(sid == 0)
    def _():
        pltpu.sync_copy(buf_a.at[pl.ds(0, 1)], acc_sh.at[pl.ds(S, 1)])
        pltpu.sync_copy(buf_a.at[pl.ds(0, 1)], cnt_sh.at[pl.ds(S, 1)])

    pltpu.sync_copy(ids_hbm.at[w], idx_v)

    plsc.subcore_barrier()

    def start_load(jl, buf, sem):
        c = w * CPW + jl
        row0 = c * CHUNK

        @pl.when((jl < CPW) & (c < FULL_CHUNKS))
        def _():
            pltpu.make_async_copy(x_hbm.at[pl.ds(row0, CHUNK)], buf, sem).start()

        @pl.when((jl < CPW) & (c == FULL_CHUNKS))
        def _():
            pltpu.make_async_copy(
                x_hbm.at[pl.ds(row0, REM)], buf.at[pl.ds(0, REM)], sem).start()

    def wait_load(jl, buf, sem):
        c = w * CPW + jl

        @pl.when((jl < CPW) & (c < FULL_CHUNKS))
        def _():
            pltpu.make_async_copy(x_hbm.at[pl.ds(0, CHUNK)], buf, sem).wait()

        @pl.when((jl < CPW) & (c == FULL_CHUNKS))
        def _():
            pltpu.make_async_copy(
                x_hbm.at[pl.ds(0, REM)], buf.at[pl.ds(0, REM)], sem).wait()

    def scatter(jl, buf):
        c = w * CPW + jl

        @pl.when((jl < CPW) & (c <= FULL_CHUNKS))
        def _():
            pltpu.sync_copy(buf, acc_sh.at[idx_v.at[jl]], add=True)
            pltpu.sync_copy(ones_v, cnt_sh.at[idx_v.at[jl]], add=True)

    start_load(0, buf_a, sem_a)

    @pl.loop(0, CPW + 1, step=2)
    def _(j):
        start_load(j + 1, buf_b, sem_b)
        wait_load(j, buf_a, sem_a)
        scatter(j, buf_a)
        start_load(j + 2, buf_a, sem_a)
        wait_load(j + 1, buf_b, sem_b)
        scatter(j + 1, buf_b)

    plsc.subcore_barrier()

    pltpu.sync_copy(acc_sh.at[pl.ds(32 * sid, 32)],
                    sums_hbm.at[cid, pl.ds(32 * sid, 32)])
    pltpu.sync_copy(cnt_sh.at[pl.ds(32 * sid, 32)],
                    cnts_hbm.at[cid, pl.ds(32 * sid, 32)])


@jax.jit
def _seg_sum(x, ids2d):
    return pl.kernel(
        _seg_sum_body,
        out_type=[
            jax.ShapeDtypeStruct((NC, S, D), jnp.float32),
            jax.ShapeDtypeStruct((NC, S, D), jnp.float32),
        ],
        mesh=plsc.VectorSubcoreMesh(
            core_axis_name="c", subcore_axis_name="s",
            num_cores=NC, num_subcores=NS),
        scratch_types=[
            pltpu.VMEM((CPW, CHUNK), jnp.int32),
            pltpu.VMEM((CHUNK, D), jnp.float32),
            pltpu.VMEM((CHUNK, D), jnp.float32),
            pltpu.VMEM((CHUNK, D), jnp.float32),
            pltpu.VMEM_SHARED((S + 1, D), jnp.float32),
            pltpu.VMEM_SHARED((S + 1, D), jnp.float32),
            pltpu.SemaphoreType.DMA,
            pltpu.SemaphoreType.DMA,
        ],
    )(x, ids2d)


def _combine_body(sums_ref, cnts_ref, out_ref):
    s = sums_ref[0] + sums_ref[1]
    c = cnts_ref[0] + cnts_ref[1]
    out_ref[...] = s / jnp.maximum(c, 1.0)


@jax.jit
def _combine(sums, cnts):
    return pl.pallas_call(
        _combine_body,
        out_shape=jax.ShapeDtypeStruct((S, D), jnp.float32),
    )(sums, cnts)


def kernel(t0, t1, t2, t3, t4, t5, t6):
    ids = t4.astype(jnp.int32)
    pad = jnp.full((TOT_CHUNKS * CHUNK - N,), S, dtype=jnp.int32)
    ids3d = jnp.concatenate([ids, pad]).reshape(NW, CPW, CHUNK)
    sums, cnts = _seg_sum(t0, ids3d)
    x_graph = _combine(sums, cnts)
    return (t0, t1, t2, t3, t4, x_graph, t6)


# speedup vs baseline: 6.5770x; 1.3911x over previous
import functools

import jax
import jax.numpy as jnp
from jax import lax
from jax.experimental import pallas as pl
from jax.experimental.pallas import tpu as pltpu
from jax.experimental.pallas import tpu_sc as plsc

N = 100000
D = 128
S = 512
NC = 2
NS = 16
NW = NC * NS
CHUNK = 128
TOT_CHUNKS = 800
CPW = TOT_CHUNKS // NW
FULL_CHUNKS = N // CHUNK
REM = N - FULL_CHUNKS * CHUNK


def _seg_sum_body(x_hbm, ids_hbm, sums_hbm, cnts_hbm, xout_hbm,
                  idx_v, buf_a, buf_b, ones_v, acc_sh, cnt_sh,
                  sem_a, sem_b, sem_wa, sem_wb):
    cid = lax.axis_index("c")
    sid = lax.axis_index("s")
    w = sid * NC + cid

    zvec = jnp.zeros((16,), jnp.float32)
    onevec = jnp.ones((16,), jnp.float32)

    @pl.loop(0, 33)
    def _(r):
        for k in range(D // 16):
            buf_a[r, pl.ds(16 * k, 16)] = zvec

    @pl.loop(0, CHUNK)
    def _(r):
        for k in range(D // 16):
            ones_v[r, pl.ds(16 * k, 16)] = onevec

    pltpu.sync_copy(buf_a.at[pl.ds(0, 32)], acc_sh.at[pl.ds(32 * sid, 32)])
    pltpu.sync_copy(buf_a.at[pl.ds(0, 32)], cnt_sh.at[pl.ds(32 * sid, 32)])

    @pl.when(sid == 0)
    def _():
        pltpu.sync_copy(buf_a.at[pl.ds(0, 1)], acc_sh.at[pl.ds(S, 1)])
        pltpu.sync_copy(buf_a.at[pl.ds(0, 1)], cnt_sh.at[pl.ds(S, 1)])

    pltpu.sync_copy(ids_hbm.at[w], idx_v)

    plsc.subcore_barrier()

    def start_load(jl, buf, sem):
        c = w * CPW + jl
        row0 = c * CHUNK

        @pl.when((jl < CPW) & (c < FULL_CHUNKS))
        def _():
            pltpu.make_async_copy(x_hbm.at[pl.ds(row0, CHUNK)], buf, sem).start()

        @pl.when((jl < CPW) & (c == FULL_CHUNKS))
        def _():
            pltpu.make_async_copy(
                x_hbm.at[pl.ds(row0, REM)], buf.at[pl.ds(0, REM)], sem).start()

    def wait_load(jl, buf, sem):
        c = w * CPW + jl

        @pl.when((jl < CPW) & (c < FULL_CHUNKS))
        def _():
            pltpu.make_async_copy(x_hbm.at[pl.ds(0, CHUNK)], buf, sem).wait()

        @pl.when((jl < CPW) & (c == FULL_CHUNKS))
        def _():
            pltpu.make_async_copy(
                x_hbm.at[pl.ds(0, REM)], buf.at[pl.ds(0, REM)], sem).wait()

    def scatter(jl, buf):
        c = w * CPW + jl

        @pl.when((jl < CPW) & (c <= FULL_CHUNKS))
        def _():
            pltpu.sync_copy(buf, acc_sh.at[idx_v.at[jl]], add=True)
            pltpu.sync_copy(ones_v, cnt_sh.at[idx_v.at[jl]], add=True)

    def start_write(jl, buf, sem):
        c = w * CPW + jl
        row0 = c * CHUNK

        @pl.when((jl < CPW) & (c < FULL_CHUNKS))
        def _():
            pltpu.make_async_copy(buf, xout_hbm.at[pl.ds(row0, CHUNK)], sem).start()

        @pl.when((jl < CPW) & (c == FULL_CHUNKS))
        def _():
            pltpu.make_async_copy(
                buf.at[pl.ds(0, REM)], xout_hbm.at[pl.ds(row0, REM)], sem).start()

    def wait_write(jl, buf, sem):
        c = w * CPW + jl

        @pl.when((jl < CPW) & (c < FULL_CHUNKS))
        def _():
            pltpu.make_async_copy(buf, xout_hbm.at[pl.ds(0, CHUNK)], sem).wait()

        @pl.when((jl < CPW) & (c == FULL_CHUNKS))
        def _():
            pltpu.make_async_copy(
                buf.at[pl.ds(0, REM)], xout_hbm.at[pl.ds(0, REM)], sem).wait()

    start_load(0, buf_a, sem_a)

    @pl.loop(0, CPW + 1, step=2)
    def _(j):
        start_load(j + 1, buf_b, sem_b)
        wait_load(j, buf_a, sem_a)
        start_write(j, buf_a, sem_wa)
        scatter(j, buf_a)
        wait_write(j, buf_a, sem_wa)
        start_load(j + 2, buf_a, sem_a)
        wait_load(j + 1, buf_b, sem_b)
        start_write(j + 1, buf_b, sem_wb)
        scatter(j + 1, buf_b)
        wait_write(j + 1, buf_b, sem_wb)

    plsc.subcore_barrier()

    pltpu.sync_copy(acc_sh.at[pl.ds(32 * sid, 32)],
                    sums_hbm.at[cid, pl.ds(32 * sid, 32)])
    pltpu.sync_copy(cnt_sh.at[pl.ds(32 * sid, 32)],
                    cnts_hbm.at[cid, pl.ds(32 * sid, 32)])


@jax.jit
def _seg_sum(x, ids2d):
    return pl.kernel(
        _seg_sum_body,
        out_type=[
            jax.ShapeDtypeStruct((NC, S, D), jnp.float32),
            jax.ShapeDtypeStruct((NC, S, D), jnp.float32),
            jax.ShapeDtypeStruct((N, D), jnp.float32),
        ],
        mesh=plsc.VectorSubcoreMesh(
            core_axis_name="c", subcore_axis_name="s",
            num_cores=NC, num_subcores=NS),
        scratch_types=[
            pltpu.VMEM((CPW, CHUNK), jnp.int32),
            pltpu.VMEM((CHUNK, D), jnp.float32),
            pltpu.VMEM((CHUNK, D), jnp.float32),
            pltpu.VMEM((CHUNK, D), jnp.float32),
            pltpu.VMEM_SHARED((S + 1, D), jnp.float32),
            pltpu.VMEM_SHARED((S + 1, D), jnp.float32),
            pltpu.SemaphoreType.DMA,
            pltpu.SemaphoreType.DMA,
            pltpu.SemaphoreType.DMA,
            pltpu.SemaphoreType.DMA,
        ],
    )(x, ids2d)


def _combine_body(sums_ref, cnts_ref, out_ref):
    s = sums_ref[0] + sums_ref[1]
    c = cnts_ref[0] + cnts_ref[1]
    out_ref[...] = s / jnp.maximum(c, 1.0)


@jax.jit
def _combine(sums, cnts):
    return pl.pallas_call(
        _combine_body,
        out_shape=jax.ShapeDtypeStruct((S, D), jnp.float32),
    )(sums, cnts)


def kernel(t0, t1, t2, t3, t4, t5, t6):
    ids = t4.astype(jnp.int32)
    pad = jnp.full((TOT_CHUNKS * CHUNK - N,), S, dtype=jnp.int32)
    ids3d = jnp.concatenate([ids, pad]).reshape(NW, CPW, CHUNK)
    sums, cnts, x_out = _seg_sum(t0, ids3d)
    x_graph = _combine(sums, cnts)
    return (x_out, t1, t2, t3, t4, x_graph, t6)


# speedup vs baseline: 6.6685x; 1.0139x over previous
import functools

import jax
import jax.numpy as jnp
from jax import lax
from jax.experimental import pallas as pl
from jax.experimental.pallas import tpu as pltpu
from jax.experimental.pallas import tpu_sc as plsc

N = 100000
D = 128
S = 512
NC = 2
NS = 16
NW = NC * NS
CHUNK = 128
TOT_CHUNKS = 800
CPW = TOT_CHUNKS // NW
FULL_CHUNKS = N // CHUNK
REM = N - FULL_CHUNKS * CHUNK
CW = 128


def _seg_sum_body(x_hbm, ids_hbm, sums_hbm, cnts_hbm, xout_hbm,
                  idx_v, buf_a, buf_b, ones_v, zc_v, acc_sh, cnt_sh,
                  sem_a, sem_b, sem_wa, sem_wb, sem_sa, sem_sb, sem_c):
    cid = lax.axis_index("c")
    sid = lax.axis_index("s")
    w = sid * NC + cid

    zvec = jnp.zeros((16,), jnp.float32)
    onevec = jnp.ones((16,), jnp.float32)

    @pl.loop(0, 33)
    def _(r):
        for k in range(D // 16):
            buf_a[r, pl.ds(16 * k, 16)] = zvec

    @pl.loop(0, CHUNK)
    def _(r):
        for k in range(CW // 16):
            ones_v[r, pl.ds(16 * k, 16)] = onevec

    @pl.loop(0, 33)
    def _(r):
        for k in range(CW // 16):
            zc_v[r, pl.ds(16 * k, 16)] = zvec

    pltpu.sync_copy(buf_a.at[pl.ds(0, 32)], acc_sh.at[pl.ds(32 * sid, 32)])
    pltpu.sync_copy(zc_v.at[pl.ds(0, 32)], cnt_sh.at[pl.ds(32 * sid, 32)])

    @pl.when(sid == 0)
    def _():
        pltpu.sync_copy(buf_a.at[pl.ds(0, 1)], acc_sh.at[pl.ds(S, 1)])
        pltpu.sync_copy(zc_v.at[pl.ds(0, 1)], cnt_sh.at[pl.ds(S, 1)])

    pltpu.sync_copy(ids_hbm.at[w], idx_v)

    plsc.subcore_barrier()

    def start_load(jl, buf, sem):
        c = w * CPW + jl
        row0 = c * CHUNK

        @pl.when((jl < CPW) & (c < FULL_CHUNKS))
        def _():
            pltpu.make_async_copy(x_hbm.at[pl.ds(row0, CHUNK)], buf, sem).start()

        @pl.when((jl < CPW) & (c == FULL_CHUNKS))
        def _():
            pltpu.make_async_copy(
                x_hbm.at[pl.ds(row0, REM)], buf.at[pl.ds(0, REM)], sem).start()

    def wait_load(jl, buf, sem):
        c = w * CPW + jl

        @pl.when((jl < CPW) & (c < FULL_CHUNKS))
        def _():
            pltpu.make_async_copy(x_hbm.at[pl.ds(0, CHUNK)], buf, sem).wait()

        @pl.when((jl < CPW) & (c == FULL_CHUNKS))
        def _():
            pltpu.make_async_copy(
                x_hbm.at[pl.ds(0, REM)], buf.at[pl.ds(0, REM)], sem).wait()

    def start_scat(jl, buf, sem):
        c = w * CPW + jl

        @pl.when((jl < CPW) & (c <= FULL_CHUNKS))
        def _():
            pltpu.async_copy(buf, acc_sh.at[idx_v.at[jl]], sem, add=True)

    def wait_scat(jl, buf, sem):
        c = w * CPW + jl

        @pl.when((jl < CPW) & (c <= FULL_CHUNKS))
        def _():
            pltpu.make_async_copy(buf, acc_sh.at[idx_v.at[jl]], sem).wait()

    def start_cnt(jl, sem):
        c = w * CPW + jl

        @pl.when((jl < CPW) & (c <= FULL_CHUNKS))
        def _():
            pltpu.async_copy(ones_v, cnt_sh.at[idx_v.at[jl]], sem, add=True)

    def wait_cnt(jl, sem):
        c = w * CPW + jl

        @pl.when((jl < CPW) & (c <= FULL_CHUNKS))
        def _():
            pltpu.make_async_copy(ones_v, cnt_sh.at[idx_v.at[jl]], sem).wait()

    def start_write(jl, buf, sem):
        c = w * CPW + jl
        row0 = c * CHUNK

        @pl.when((jl < CPW) & (c < FULL_CHUNKS))
        def _():
            pltpu.make_async_copy(buf, xout_hbm.at[pl.ds(row0, CHUNK)], sem).start()

        @pl.when((jl < CPW) & (c == FULL_CHUNKS))
        def _():
            pltpu.make_async_copy(
                buf.at[pl.ds(0, REM)], xout_hbm.at[pl.ds(row0, REM)], sem).start()

    def wait_write(jl, buf, sem):
        c = w * CPW + jl

        @pl.when((jl < CPW) & (c < FULL_CHUNKS))
        def _():
            pltpu.make_async_copy(buf, xout_hbm.at[pl.ds(0, CHUNK)], sem).wait()

        @pl.when((jl < CPW) & (c == FULL_CHUNKS))
        def _():
            pltpu.make_async_copy(
                buf.at[pl.ds(0, REM)], xout_hbm.at[pl.ds(0, REM)], sem).wait()

    start_load(0, buf_a, sem_a)
    start_load(1, buf_b, sem_b)

    @pl.loop(0, CPW + 1, step=2)
    def _(j):
        wait_load(j, buf_a, sem_a)
        start_write(j, buf_a, sem_wa)
        start_scat(j, buf_a, sem_sa)
        start_cnt(j, sem_c)
        wait_load(j + 1, buf_b, sem_b)
        start_write(j + 1, buf_b, sem_wb)
        start_scat(j + 1, buf_b, sem_sb)
        start_cnt(j + 1, sem_c)
        wait_write(j, buf_a, sem_wa)
        wait_scat(j, buf_a, sem_sa)
        start_load(j + 2, buf_a, sem_a)
        wait_write(j + 1, buf_b, sem_wb)
        wait_scat(j + 1, buf_b, sem_sb)
        start_load(j + 3, buf_b, sem_b)

    @pl.loop(0, CPW + 1)
    def _(j):
        wait_cnt(j, sem_c)

    plsc.subcore_barrier()

    pltpu.sync_copy(acc_sh.at[pl.ds(32 * sid, 32)],
                    sums_hbm.at[cid, pl.ds(32 * sid, 32)])
    pltpu.sync_copy(cnt_sh.at[pl.ds(32 * sid, 32)],
                    cnts_hbm.at[cid, pl.ds(32 * sid, 32)])


@jax.jit
def _seg_sum(x, ids2d):
    return pl.kernel(
        _seg_sum_body,
        out_type=[
            jax.ShapeDtypeStruct((NC, S, D), jnp.float32),
            jax.ShapeDtypeStruct((NC, S, CW), jnp.float32),
            jax.ShapeDtypeStruct((N, D), jnp.float32),
        ],
        mesh=plsc.VectorSubcoreMesh(
            core_axis_name="c", subcore_axis_name="s",
            num_cores=NC, num_subcores=NS),
        scratch_types=[
            pltpu.VMEM((CPW, CHUNK), jnp.int32),
            pltpu.VMEM((CHUNK, D), jnp.float32),
            pltpu.VMEM((CHUNK, D), jnp.float32),
            pltpu.VMEM((CHUNK, CW), jnp.float32),
            pltpu.VMEM((33, CW), jnp.float32),
            pltpu.VMEM_SHARED((S + 1, D), jnp.float32),
            pltpu.VMEM_SHARED((S + 1, CW), jnp.float32),
            pltpu.SemaphoreType.DMA,
            pltpu.SemaphoreType.DMA,
            pltpu.SemaphoreType.DMA,
            pltpu.SemaphoreType.DMA,
            pltpu.SemaphoreType.DMA,
            pltpu.SemaphoreType.DMA,
            pltpu.SemaphoreType.DMA,
        ],
    )(x, ids2d)


def _combine_body(sums_ref, cnts_ref, out_ref):
    s = sums_ref[0] + sums_ref[1]
    c = cnts_ref[0, :, 0:1] + cnts_ref[1, :, 0:1]
    out_ref[...] = s / jnp.maximum(c, 1.0)


@jax.jit
def _combine(sums, cnts):
    return pl.pallas_call(
        _combine_body,
        out_shape=jax.ShapeDtypeStruct((S, D), jnp.float32),
    )(sums, cnts)


def kernel(t0, t1, t2, t3, t4, t5, t6):
    ids = t4.astype(jnp.int32)
    pad = jnp.full((TOT_CHUNKS * CHUNK - N,), S, dtype=jnp.int32)
    ids3d = jnp.concatenate([ids, pad]).reshape(NW, CPW, CHUNK)
    sums, cnts, x_out = _seg_sum(t0, ids3d)
    x_graph = _combine(sums, cnts)
    return (x_out, t1, t2, t3, t4, x_graph, t6)
